# baseline (speedup 1.0000x reference)
import jax
import jax.numpy as jnp
from jax import lax
from jax.experimental import pallas as pl
from jax.experimental.pallas import tpu as pltpu

_GRP = 16
_TB = 512


def _dm_gather_kernel(ids_ref, x0_ref, w_ref, wp_ref, out_ref, ws_ref, g_ref):
    tb, s_dim = out_ref.shape
    n_ids = ids_ref.shape[1]
    c_dim = n_ids - s_dim

    def group(g, carry):
        base = pl.multiple_of(g * _GRP, _GRP)
        for j in range(_GRP):
            b = base + j
            acc = w_ref[ids_ref[b, 0], 0]
            for c in range(1, c_dim):
                acc = acc + w_ref[ids_ref[b, c], 0]
            ws_ref[j] = acc
        for s in range(s_dim):
            for j in range(_GRP):
                b = base + j
                g_ref[s * _GRP + j] = wp_ref[ids_ref[b, c_dim + s], 0]
        xg = x0_ref[pl.ds(base, _GRP), :] + ws_ref[...]
        cols = []
        for s in range(s_dim):
            gs = g_ref[s * _GRP:(s + 1) * _GRP, :]
            cols.append(jnp.sum(xg * gs, axis=1, keepdims=True))
        out_ref[pl.ds(base, _GRP), :] = jnp.concatenate(cols, axis=1)
        return carry

    lax.fori_loop(0, tb // _GRP, group, 0)


def kernel(D, W, Wp, ctx_ids, doc_ids, target_and_noise_ids):
    B, C = ctx_ids.shape
    S = target_and_noise_ids.shape[1]
    E, Nw = Wp.shape
    Nwp1 = W.shape[0]

    x0 = D.astype(jnp.float32)[doc_ids]
    ids = jnp.concatenate(
        [ctx_ids.astype(jnp.int32),
         target_and_noise_ids.astype(jnp.int32)], axis=1)

    w3 = W.astype(jnp.float32).reshape(Nwp1, 1, E)
    wpt = Wp.astype(jnp.float32).T.reshape(Nw, 1, E)

    n_blocks = pl.cdiv(B, _TB)
    b_pad = n_blocks * _TB - B
    if b_pad:
        x0 = jnp.pad(x0, ((0, b_pad), (0, 0)))
        ids = jnp.pad(ids, ((0, b_pad), (0, 0)))

    out = pl.pallas_call(
        _dm_gather_kernel,
        out_shape=jax.ShapeDtypeStruct((n_blocks * _TB, S), jnp.float32),
        grid=(n_blocks,),
        in_specs=[
            pl.BlockSpec((_TB, C + S), lambda b: (b, 0),
                         memory_space=pltpu.SMEM),
            pl.BlockSpec((_TB, E), lambda b: (b, 0)),
            pl.BlockSpec((Nwp1, 1, E), lambda b: (0, 0, 0)),
            pl.BlockSpec((Nw, 1, E), lambda b: (0, 0, 0)),
        ],
        out_specs=pl.BlockSpec((_TB, S), lambda b: (b, 0)),
        scratch_shapes=[
            pltpu.VMEM((_GRP, E), jnp.float32),
            pltpu.VMEM((S * _GRP, E), jnp.float32),
        ],
        compiler_params=pltpu.CompilerParams(
            dimension_semantics=("arbitrary",),
            vmem_limit_bytes=46 * 1024 * 1024,
        ),
    )(ids, x0, w3, wpt)

    return out[:B]


# speedup vs baseline: 1.0714x; 1.0714x over previous
import jax
import jax.numpy as jnp
from jax import lax
from jax.experimental import pallas as pl
from jax.experimental.pallas import tpu as pltpu

_GRP = 16
_TB = 512


def _dm_gather_kernel(ids_ref, x0_ref, w_ref, wp_ref, out_ref, ws_ref, g_ref):
    tb, s_dim = out_ref.shape
    n_ids = ids_ref.shape[1]
    c_dim = n_ids - s_dim

    def group(gg, carry):
        for p in range(2):
            base = pl.multiple_of(gg * 2 * _GRP + p * _GRP, _GRP)
            ws0 = p * _GRP
            g0 = p * s_dim * _GRP
            for j in range(_GRP):
                b = base + j
                acc = w_ref[ids_ref[b, 0], 0]
                for c in range(1, c_dim):
                    acc = acc + w_ref[ids_ref[b, c], 0]
                ws_ref[ws0 + j] = acc
            for s in range(s_dim):
                for j in range(_GRP):
                    b = base + j
                    g_ref[g0 + s * _GRP + j] = wp_ref[ids_ref[b, c_dim + s], 0]
            xg = x0_ref[pl.ds(base, _GRP), :] + ws_ref[ws0:ws0 + _GRP, :]
            cols = []
            for s in range(s_dim):
                gs = g_ref[g0 + s * _GRP:g0 + (s + 1) * _GRP, :]
                cols.append(jnp.sum(xg * gs, axis=1, keepdims=True))
            out_ref[pl.ds(base, _GRP), :] = jnp.concatenate(cols, axis=1)
        return carry

    lax.fori_loop(0, tb // (2 * _GRP), group, 0)


def kernel(D, W, Wp, ctx_ids, doc_ids, target_and_noise_ids):
    B, C = ctx_ids.shape
    S = target_and_noise_ids.shape[1]
    E, Nw = Wp.shape
    Nwp1 = W.shape[0]

    x0 = D.astype(jnp.float32)[doc_ids]
    ids = jnp.concatenate(
        [ctx_ids.astype(jnp.int32),
         target_and_noise_ids.astype(jnp.int32)], axis=1)

    w3 = W.astype(jnp.float32).reshape(Nwp1, 1, E)
    wpt = Wp.astype(jnp.float32).T.reshape(Nw, 1, E)

    n_blocks = pl.cdiv(B, _TB)
    b_pad = n_blocks * _TB - B
    if b_pad:
        x0 = jnp.pad(x0, ((0, b_pad), (0, 0)))
        ids = jnp.pad(ids, ((0, b_pad), (0, 0)))

    out = pl.pallas_call(
        _dm_gather_kernel,
        out_shape=jax.ShapeDtypeStruct((n_blocks * _TB, S), jnp.float32),
        grid=(n_blocks,),
        in_specs=[
            pl.BlockSpec((_TB, C + S), lambda b: (b, 0),
                         memory_space=pltpu.SMEM),
            pl.BlockSpec((_TB, E), lambda b: (b, 0)),
            pl.BlockSpec((Nwp1, 1, E), lambda b: (0, 0, 0)),
            pl.BlockSpec((Nw, 1, E), lambda b: (0, 0, 0)),
        ],
        out_specs=pl.BlockSpec((_TB, S), lambda b: (b, 0)),
        scratch_shapes=[
            pltpu.VMEM((2 * _GRP, E), jnp.float32),
            pltpu.VMEM((2 * S * _GRP, E), jnp.float32),
        ],
        compiler_params=pltpu.CompilerParams(
            dimension_semantics=("arbitrary",),
            vmem_limit_bytes=46 * 1024 * 1024,
        ),
    )(ids, x0, w3, wpt)

    return out[:B]


# speedup vs baseline: 1.0913x; 1.0186x over previous
import jax
import jax.numpy as jnp
from jax import lax
from jax.experimental import pallas as pl
from jax.experimental.pallas import tpu as pltpu

_GRP = 16
_TB = 256


def _dm_gather_kernel(cids_ref, tids_ref, x0_ref, w_ref, wp_ref, out_ref,
                      ws_ref, g_ref):
    tb, s_dim = out_ref.shape
    c_dim = cids_ref.shape[1]

    def group(gg, carry):
        for p in range(2):
            base = pl.multiple_of(gg * 2 * _GRP + p * _GRP, _GRP)
            ws0 = p * _GRP
            g0 = p * s_dim * _GRP
            for j in range(_GRP):
                b = base + j
                acc = w_ref[cids_ref[b, 0], 0]
                for c in range(1, c_dim):
                    acc = acc + w_ref[cids_ref[b, c], 0]
                ws_ref[ws0 + j] = acc
            for s in range(s_dim):
                for j in range(_GRP):
                    b = base + j
                    g_ref[g0 + s * _GRP + j] = wp_ref[tids_ref[b, s], 0]
            xg = x0_ref[pl.ds(base, _GRP), :] + ws_ref[ws0:ws0 + _GRP, :]
            cols = []
            for s in range(s_dim):
                gs = g_ref[g0 + s * _GRP:g0 + (s + 1) * _GRP, :]
                cols.append(jnp.sum(xg * gs, axis=1, keepdims=True))
            out_ref[pl.ds(base, _GRP), :] = jnp.concatenate(cols, axis=1)
        return carry

    lax.fori_loop(0, tb // (2 * _GRP), group, 0)


def kernel(D, W, Wp, ctx_ids, doc_ids, target_and_noise_ids):
    B, C = ctx_ids.shape
    S = target_and_noise_ids.shape[1]
    E, Nw = Wp.shape
    Nwp1 = W.shape[0]

    x0 = D.astype(jnp.float32)[doc_ids]
    cids = ctx_ids.astype(jnp.int32)
    tids = target_and_noise_ids.astype(jnp.int32)

    w3 = W.astype(jnp.float32).reshape(Nwp1, 1, E)
    wpt = Wp.astype(jnp.float32).T.reshape(Nw, 1, E)

    n_blocks = pl.cdiv(B, _TB)
    b_pad = n_blocks * _TB - B
    if b_pad:
        x0 = jnp.pad(x0, ((0, b_pad), (0, 0)))
        cids = jnp.pad(cids, ((0, b_pad), (0, 0)))
        tids = jnp.pad(tids, ((0, b_pad), (0, 0)))

    out = pl.pallas_call(
        _dm_gather_kernel,
        out_shape=jax.ShapeDtypeStruct((n_blocks * _TB, S), jnp.float32),
        grid=(n_blocks,),
        in_specs=[
            pl.BlockSpec((_TB, C), lambda b: (b, 0),
                         memory_space=pltpu.SMEM),
            pl.BlockSpec((_TB, S), lambda b: (b, 0),
                         memory_space=pltpu.SMEM),
            pl.BlockSpec((_TB, E), lambda b: (b, 0)),
            pl.BlockSpec((Nwp1, 1, E), lambda b: (0, 0, 0)),
            pl.BlockSpec((Nw, 1, E), lambda b: (0, 0, 0)),
        ],
        out_specs=pl.BlockSpec((_TB, S), lambda b: (b, 0)),
        scratch_shapes=[
            pltpu.VMEM((2 * _GRP, E), jnp.float32),
            pltpu.VMEM((2 * S * _GRP, E), jnp.float32),
        ],
        compiler_params=pltpu.CompilerParams(
            dimension_semantics=("arbitrary",),
            vmem_limit_bytes=46 * 1024 * 1024,
        ),
    )(cids, tids, x0, w3, wpt)

    return out[:B]


# speedup vs baseline: 1.1296x; 1.0351x over previous
import jax
import jax.numpy as jnp
from jax import lax
from jax.experimental import pallas as pl
from jax.experimental.pallas import tpu as pltpu

_GRP = 16
_ROT = 2
_TB = 256


def _rup(v, m):
    return -(-v // m) * m


def _dm_gather_kernel(cids_ref, tids_ref, x0_ref, w2_ref, wp2_ref, out_ref,
                      ws_ref, g_ref, w3_ref, wpt_ref):
    tb, s_dim = out_ref.shape
    c_dim = cids_ref.shape[1]
    e_dim = x0_ref.shape[1]

    @pl.when(pl.program_id(0) == 0)
    def _prep_tables():
        def wcopy(k, c):
            r = pl.multiple_of(k * 8, 8)
            w3_ref[pl.ds(r, 8)] = w2_ref[pl.ds(r, 8), :].reshape(8, 1, e_dim)
            return c
        lax.fori_loop(0, w3_ref.shape[0] // 8, wcopy, 0)
        ii = lax.broadcasted_iota(jnp.int32, (e_dim, e_dim), 0)
        jj = lax.broadcasted_iota(jnp.int32, (e_dim, e_dim), 1)
        eye = jnp.where(ii == jj, 1.0, 0.0).astype(jnp.float32)
        for k in range(wpt_ref.shape[0] // e_dim):
            chunk = wp2_ref[:, k * e_dim:(k + 1) * e_dim]
            t = lax.dot_general(eye, chunk, (((1,), (1,)), ((), ())),
                                preferred_element_type=jnp.float32)
            wpt_ref[k * e_dim:(k + 1) * e_dim] = t.reshape(e_dim, 1, e_dim)

    def group(gg, carry):
        for p in range(_ROT):
            base = pl.multiple_of(gg * _ROT * _GRP + p * _GRP, _GRP)
            ws0 = p * _GRP
            g0 = p * s_dim * _GRP
            for j in range(_GRP):
                b = base + j
                acc = w3_ref[cids_ref[b, 0], 0]
                for c in range(1, c_dim):
                    acc = acc + w3_ref[cids_ref[b, c], 0]
                ws_ref[ws0 + j] = acc
            for s in range(s_dim):
                for j in range(_GRP):
                    b = base + j
                    g_ref[g0 + s * _GRP + j] = wpt_ref[tids_ref[b, s], 0]
            xg = x0_ref[pl.ds(base, _GRP), :] + ws_ref[ws0:ws0 + _GRP, :]
            cols = []
            for s in range(s_dim):
                gs = g_ref[g0 + s * _GRP:g0 + (s + 1) * _GRP, :]
                cols.append(jnp.sum(xg * gs, axis=1, keepdims=True))
            out_ref[pl.ds(base, _GRP), :] = jnp.concatenate(cols, axis=1)
        return carry

    lax.fori_loop(0, tb // (_ROT * _GRP), group, 0)


def kernel(D, W, Wp, ctx_ids, doc_ids, target_and_noise_ids):
    B, C = ctx_ids.shape
    S = target_and_noise_ids.shape[1]
    E, Nw = Wp.shape
    Nwp1 = W.shape[0]

    x0 = D.astype(jnp.float32)[doc_ids]
    cids = ctx_ids.astype(jnp.int32)
    tids = target_and_noise_ids.astype(jnp.int32)
    w2 = W.astype(jnp.float32)
    wp2 = Wp.astype(jnp.float32)

    nw_rows = _rup(Nwp1, 8)
    nt_rows = _rup(Nw, E)

    n_blocks = pl.cdiv(B, _TB)
    b_pad = n_blocks * _TB - B
    if b_pad:
        x0 = jnp.pad(x0, ((0, b_pad), (0, 0)))
        cids = jnp.pad(cids, ((0, b_pad), (0, 0)))
        tids = jnp.pad(tids, ((0, b_pad), (0, 0)))

    out = pl.pallas_call(
        _dm_gather_kernel,
        out_shape=jax.ShapeDtypeStruct((n_blocks * _TB, S), jnp.float32),
        grid=(n_blocks,),
        in_specs=[
            pl.BlockSpec((_TB, C), lambda b: (b, 0),
                         memory_space=pltpu.SMEM),
            pl.BlockSpec((_TB, S), lambda b: (b, 0),
                         memory_space=pltpu.SMEM),
            pl.BlockSpec((_TB, E), lambda b: (b, 0)),
            pl.BlockSpec((nw_rows, E), lambda b: (0, 0)),
            pl.BlockSpec((E, nt_rows), lambda b: (0, 0)),
        ],
        out_specs=pl.BlockSpec((_TB, S), lambda b: (b, 0)),
        scratch_shapes=[
            pltpu.VMEM((_ROT * _GRP, E), jnp.float32),
            pltpu.VMEM((_ROT * S * _GRP, E), jnp.float32),
            pltpu.VMEM((nw_rows, 1, E), jnp.float32),
            pltpu.VMEM((nt_rows, 1, E), jnp.float32),
        ],
        compiler_params=pltpu.CompilerParams(
            dimension_semantics=("arbitrary",),
            vmem_limit_bytes=46 * 1024 * 1024,
        ),
    )(cids, tids, x0, w2, wp2)

    return out[:B]


# speedup vs baseline: 1.1366x; 1.0061x over previous
import jax
import jax.numpy as jnp
from jax import lax
from jax.experimental import pallas as pl
from jax.experimental.pallas import tpu as pltpu

_GRP = 16
_ROT = 2
_TB = 256


def _rup(v, m):
    return -(-v // m) * m


def _dm_gather_kernel(cids_ref, tids_ref, x0_ref, w2_ref, wp2_ref, out_ref,
                      ws_ref, g_ref, w3_ref, wpt_ref):
    tb, s_dim = out_ref.shape
    c_dim = cids_ref.shape[1]
    e_dim = x0_ref.shape[1]

    @pl.when(pl.program_id(0) == 0)
    def _prep_tables():
        def wcopy(k, c):
            r = pl.multiple_of(k * 8, 8)
            w3_ref[pl.ds(r, 8)] = w2_ref[pl.ds(r, 8), :].reshape(8, 1, e_dim)
            return c
        lax.fori_loop(0, w3_ref.shape[0] // 8, wcopy, 0)
        ii = lax.broadcasted_iota(jnp.int32, (e_dim, e_dim), 0)
        jj = lax.broadcasted_iota(jnp.int32, (e_dim, e_dim), 1)
        eye = jnp.where(ii == jj, 1.0, 0.0).astype(jnp.float32)

        def tcopy(k, c):
            r = pl.multiple_of(k * e_dim, e_dim)
            chunk = wp2_ref[:, pl.ds(r, e_dim)]
            t = lax.dot_general(eye, chunk, (((1,), (1,)), ((), ())),
                                preferred_element_type=jnp.float32)
            wpt_ref[pl.ds(r, e_dim)] = t.reshape(e_dim, 1, e_dim)
            return c
        lax.fori_loop(0, wpt_ref.shape[0] // e_dim, tcopy, 0)

    def group(gg, carry):
        for p in range(_ROT):
            base = pl.multiple_of(gg * _ROT * _GRP + p * _GRP, _GRP)
            ws0 = p * _GRP
            g0 = p * s_dim * _GRP
            for j in range(_GRP):
                b = base + j
                acc = w3_ref[cids_ref[b, 0], 0]
                for c in range(1, c_dim):
                    acc = acc + w3_ref[cids_ref[b, c], 0]
                ws_ref[ws0 + j] = acc
            for s in range(s_dim):
                for j in range(_GRP):
                    b = base + j
                    g_ref[g0 + s * _GRP + j] = wpt_ref[tids_ref[b, s], 0]
            xg = x0_ref[pl.ds(base, _GRP), :] + ws_ref[ws0:ws0 + _GRP, :]
            cols = []
            for s in range(s_dim):
                gs = g_ref[g0 + s * _GRP:g0 + (s + 1) * _GRP, :]
                cols.append(jnp.sum(xg * gs, axis=1, keepdims=True))
            out_ref[pl.ds(base, _GRP), :] = jnp.concatenate(cols, axis=1)
        return carry

    lax.fori_loop(0, tb // (_ROT * _GRP), group, 0)


def kernel(D, W, Wp, ctx_ids, doc_ids, target_and_noise_ids):
    B, C = ctx_ids.shape
    S = target_and_noise_ids.shape[1]
    E, Nw = Wp.shape
    Nwp1 = W.shape[0]

    x0 = D.astype(jnp.float32)[doc_ids]
    cids = ctx_ids.astype(jnp.int32)
    tids = target_and_noise_ids.astype(jnp.int32)
    w2 = W.astype(jnp.float32)
    wp2 = Wp.astype(jnp.float32)

    nw_rows = _rup(Nwp1, 8)
    nt_rows = _rup(Nw, E)

    n_blocks = pl.cdiv(B, _TB)
    b_pad = n_blocks * _TB - B
    if b_pad:
        x0 = jnp.pad(x0, ((0, b_pad), (0, 0)))
        cids = jnp.pad(cids, ((0, b_pad), (0, 0)))
        tids = jnp.pad(tids, ((0, b_pad), (0, 0)))

    out = pl.pallas_call(
        _dm_gather_kernel,
        out_shape=jax.ShapeDtypeStruct((n_blocks * _TB, S), jnp.float32),
        grid=(n_blocks,),
        in_specs=[
            pl.BlockSpec((_TB, C), lambda b: (b, 0),
                         memory_space=pltpu.SMEM),
            pl.BlockSpec((_TB, S), lambda b: (b, 0),
                         memory_space=pltpu.SMEM),
            pl.BlockSpec((_TB, E), lambda b: (b, 0)),
            pl.BlockSpec((nw_rows, E), lambda b: (0, 0)),
            pl.BlockSpec((E, nt_rows), lambda b: (0, 0)),
        ],
        out_specs=pl.BlockSpec((_TB, S), lambda b: (b, 0)),
        scratch_shapes=[
            pltpu.VMEM((_ROT * _GRP, E), jnp.float32),
            pltpu.VMEM((_ROT * S * _GRP, E), jnp.float32),
            pltpu.VMEM((nw_rows, 1, E), jnp.float32),
            pltpu.VMEM((nt_rows, 1, E), jnp.float32),
        ],
        compiler_params=pltpu.CompilerParams(
            dimension_semantics=("arbitrary",),
            vmem_limit_bytes=46 * 1024 * 1024,
        ),
    )(cids, tids, x0, w2, wp2)

    return out[:B]


# speedup vs baseline: 1.1665x; 1.0263x over previous
import jax
import jax.numpy as jnp
from jax import lax
from jax.experimental import pallas as pl
from jax.experimental.pallas import tpu as pltpu

_GRP = 16
_ROT = 4
_TB = 256


def _rup(v, m):
    return -(-v // m) * m


def _dm_gather_kernel(cids_ref, tids_ref, x0_ref, w2_ref, wp2_ref, out_ref,
                      ws_ref, g_ref, w3_ref, wpt_ref):
    tb, s_dim = out_ref.shape
    c_dim = cids_ref.shape[1]
    e_dim = x0_ref.shape[1]

    @pl.when(pl.program_id(0) == 0)
    def _prep_tables():
        def wcopy(k, c):
            r = pl.multiple_of(k * 8, 8)
            w3_ref[pl.ds(r, 8)] = w2_ref[pl.ds(r, 8), :].reshape(8, 1, e_dim)
            return c
        lax.fori_loop(0, w3_ref.shape[0] // 8, wcopy, 0)
        ii = lax.broadcasted_iota(jnp.int32, (e_dim, e_dim), 0)
        jj = lax.broadcasted_iota(jnp.int32, (e_dim, e_dim), 1)
        eye = jnp.where(ii == jj, 1.0, 0.0).astype(jnp.float32)

        def tcopy(k, c):
            r = pl.multiple_of(k * e_dim, e_dim)
            chunk = wp2_ref[:, pl.ds(r, e_dim)]
            t = lax.dot_general(eye, chunk, (((1,), (1,)), ((), ())),
                                preferred_element_type=jnp.float32)
            wpt_ref[pl.ds(r, e_dim)] = t.reshape(e_dim, 1, e_dim)
            return c
        lax.fori_loop(0, wpt_ref.shape[0] // e_dim, tcopy, 0)

    def group(gg, carry):
        for p in range(_ROT):
            base = pl.multiple_of(gg * _ROT * _GRP + p * _GRP, _GRP)
            ws0 = p * _GRP
            g0 = p * s_dim * _GRP
            for j in range(_GRP):
                b = base + j
                acc = w3_ref[cids_ref[b, 0], 0]
                for c in range(1, c_dim):
                    acc = acc + w3_ref[cids_ref[b, c], 0]
                ws_ref[ws0 + j] = acc
            for j in range(_GRP):
                b = base + j
                for s in range(s_dim):
                    g_ref[g0 + s * _GRP + j] = wpt_ref[tids_ref[b, s], 0]
            xg = x0_ref[pl.ds(base, _GRP), :] + ws_ref[ws0:ws0 + _GRP, :]
            cols = []
            for s in range(s_dim):
                gs = g_ref[g0 + s * _GRP:g0 + (s + 1) * _GRP, :]
                cols.append(jnp.sum(xg * gs, axis=1, keepdims=True))
            out_ref[pl.ds(base, _GRP), :] = jnp.concatenate(cols, axis=1)
        return carry

    lax.fori_loop(0, tb // (_ROT * _GRP), group, 0)


def kernel(D, W, Wp, ctx_ids, doc_ids, target_and_noise_ids):
    B, C = ctx_ids.shape
    S = target_and_noise_ids.shape[1]
    E, Nw = Wp.shape
    Nwp1 = W.shape[0]

    x0 = D.astype(jnp.float32)[doc_ids]
    cids = ctx_ids.astype(jnp.int32)
    tids = target_and_noise_ids.astype(jnp.int32)
    w2 = W.astype(jnp.float32)
    wp2 = Wp.astype(jnp.float32)

    nw_rows = _rup(Nwp1, 8)
    nt_rows = _rup(Nw, E)

    n_blocks = pl.cdiv(B, _TB)
    b_pad = n_blocks * _TB - B
    if b_pad:
        x0 = jnp.pad(x0, ((0, b_pad), (0, 0)))
        cids = jnp.pad(cids, ((0, b_pad), (0, 0)))
        tids = jnp.pad(tids, ((0, b_pad), (0, 0)))

    out = pl.pallas_call(
        _dm_gather_kernel,
        out_shape=jax.ShapeDtypeStruct((n_blocks * _TB, S), jnp.float32),
        grid=(n_blocks,),
        in_specs=[
            pl.BlockSpec((_TB, C), lambda b: (b, 0),
                         memory_space=pltpu.SMEM),
            pl.BlockSpec((_TB, S), lambda b: (b, 0),
                         memory_space=pltpu.SMEM),
            pl.BlockSpec((_TB, E), lambda b: (b, 0)),
            pl.BlockSpec((nw_rows, E), lambda b: (0, 0)),
            pl.BlockSpec((E, nt_rows), lambda b: (0, 0)),
        ],
        out_specs=pl.BlockSpec((_TB, S), lambda b: (b, 0)),
        scratch_shapes=[
            pltpu.VMEM((_ROT * _GRP, E), jnp.float32),
            pltpu.VMEM((_ROT * S * _GRP, E), jnp.float32),
            pltpu.VMEM((nw_rows, 1, E), jnp.float32),
            pltpu.VMEM((nt_rows, 1, E), jnp.float32),
        ],
        compiler_params=pltpu.CompilerParams(
            dimension_semantics=("arbitrary",),
            vmem_limit_bytes=46 * 1024 * 1024,
        ),
    )(cids, tids, x0, w2, wp2)

    return out[:B]


# speedup vs baseline: 1.1826x; 1.0139x over previous
import jax
import jax.numpy as jnp
from jax import lax
from jax.experimental import pallas as pl
from jax.experimental.pallas import tpu as pltpu

_GRP = 8
_ROT = 8
_TB = 256


def _rup(v, m):
    return -(-v // m) * m


def _dm_gather_kernel(cids_ref, tids_ref, x0_ref, w2_ref, wp2_ref, out_ref,
                      ws_ref, g_ref, w3_ref, wpt_ref):
    tb, s_dim = out_ref.shape
    c_dim = cids_ref.shape[1]
    e_dim = x0_ref.shape[1]

    @pl.when(pl.program_id(0) == 0)
    def _prep_tables():
        def wcopy(k, c):
            r = pl.multiple_of(k * 8, 8)
            w3_ref[pl.ds(r, 8)] = w2_ref[pl.ds(r, 8), :].reshape(8, 1, e_dim)
            return c
        lax.fori_loop(0, w3_ref.shape[0] // 8, wcopy, 0)
        ii = lax.broadcasted_iota(jnp.int32, (e_dim, e_dim), 0)
        jj = lax.broadcasted_iota(jnp.int32, (e_dim, e_dim), 1)
        eye = jnp.where(ii == jj, 1.0, 0.0).astype(jnp.float32)

        def tcopy(k, c):
            r = pl.multiple_of(k * e_dim, e_dim)
            chunk = wp2_ref[:, pl.ds(r, e_dim)]
            t = lax.dot_general(eye, chunk, (((1,), (1,)), ((), ())),
                                preferred_element_type=jnp.float32)
            wpt_ref[pl.ds(r, e_dim)] = t.reshape(e_dim, 1, e_dim)
            return c
        lax.fori_loop(0, wpt_ref.shape[0] // e_dim, tcopy, 0)

    def group(gg, carry):
        for p in range(_ROT):
            base = pl.multiple_of(gg * _ROT * _GRP + p * _GRP, _GRP)
            ws0 = p * _GRP
            g0 = p * s_dim * _GRP
            for j in range(_GRP):
                b = base + j
                acc = w3_ref[cids_ref[b, 0], 0]
                for c in range(1, c_dim):
                    acc = acc + w3_ref[cids_ref[b, c], 0]
                ws_ref[ws0 + j] = acc
            for j in range(_GRP):
                b = base + j
                for s in range(s_dim):
                    g_ref[g0 + s * _GRP + j] = wpt_ref[tids_ref[b, s], 0]
            xg = x0_ref[pl.ds(base, _GRP), :] + ws_ref[ws0:ws0 + _GRP, :]
            cols = []
            for s in range(s_dim):
                gs = g_ref[g0 + s * _GRP:g0 + (s + 1) * _GRP, :]
                cols.append(jnp.sum(xg * gs, axis=1, keepdims=True))
            out_ref[pl.ds(base, _GRP), :] = jnp.concatenate(cols, axis=1)
        return carry

    lax.fori_loop(0, tb // (_ROT * _GRP), group, 0)


def kernel(D, W, Wp, ctx_ids, doc_ids, target_and_noise_ids):
    B, C = ctx_ids.shape
    S = target_and_noise_ids.shape[1]
    E, Nw = Wp.shape
    Nwp1 = W.shape[0]

    x0 = D.astype(jnp.float32)[doc_ids]
    cids = ctx_ids.astype(jnp.int32)
    tids = target_and_noise_ids.astype(jnp.int32)
    w2 = W.astype(jnp.float32)
    wp2 = Wp.astype(jnp.float32)

    nw_rows = _rup(Nwp1, 8)
    nt_rows = _rup(Nw, E)

    n_blocks = pl.cdiv(B, _TB)
    b_pad = n_blocks * _TB - B
    if b_pad:
        x0 = jnp.pad(x0, ((0, b_pad), (0, 0)))
        cids = jnp.pad(cids, ((0, b_pad), (0, 0)))
        tids = jnp.pad(tids, ((0, b_pad), (0, 0)))

    out = pl.pallas_call(
        _dm_gather_kernel,
        out_shape=jax.ShapeDtypeStruct((n_blocks * _TB, S), jnp.float32),
        grid=(n_blocks,),
        in_specs=[
            pl.BlockSpec((_TB, C), lambda b: (b, 0),
                         memory_space=pltpu.SMEM),
            pl.BlockSpec((_TB, S), lambda b: (b, 0),
                         memory_space=pltpu.SMEM),
            pl.BlockSpec((_TB, E), lambda b: (b, 0)),
            pl.BlockSpec((nw_rows, E), lambda b: (0, 0)),
            pl.BlockSpec((E, nt_rows), lambda b: (0, 0)),
        ],
        out_specs=pl.BlockSpec((_TB, S), lambda b: (b, 0)),
        scratch_shapes=[
            pltpu.VMEM((_ROT * _GRP, E), jnp.float32),
            pltpu.VMEM((_ROT * S * _GRP, E), jnp.float32),
            pltpu.VMEM((nw_rows, 1, E), jnp.float32),
            pltpu.VMEM((nt_rows, 1, E), jnp.float32),
        ],
        compiler_params=pltpu.CompilerParams(
            dimension_semantics=("arbitrary",),
            vmem_limit_bytes=46 * 1024 * 1024,
        ),
    )(cids, tids, x0, w2, wp2)

    return out[:B]
